# baseline (speedup 1.0000x reference)
import functools

import jax
import jax.numpy as jnp
from jax import lax
from jax.experimental import pallas as pl
from jax.experimental.pallas import tpu as pltpu
from jax.experimental.pallas import tpu_sc as plsc

LENGTH = 100
EMB = 1024
HID = 1024
ROWS = 128
TBL_WORDS = LENGTH * EMB

NC = 2
NS = 16
NW = NC * NS

N_TOKENS = 1024 * 50
B_PER_W = N_TOKENS // NW
GROUP = 8
N_GROUPS = B_PER_W // GROUP


def _tc_table_mlp(ids_ref, emb_ref, w1_ref, b1_ref, w2_ref, b2_ref, out_ref):
    ids = ids_ref[...]
    v = lax.broadcasted_iota(jnp.int32, (ROWS, ROWS), 0)
    j = lax.broadcasted_iota(jnp.int32, (ROWS, ROWS), 1)
    match = (ids == v).astype(jnp.float32)
    tri = (lax.broadcasted_iota(jnp.int32, (ROWS, ROWS), 0) <= j).astype(
        jnp.float32)
    c = jnp.dot(match, tri, preferred_element_type=jnp.float32)
    first = match * (c == 1.0).astype(jnp.float32)
    nomatch = (c[:, ROWS - 1:ROWS] == 0.0).astype(jnp.float32)
    col0 = (j == 0).astype(jnp.float32)
    sel = first + nomatch * col0
    rows = jnp.dot(sel, emb_ref[...], preferred_element_type=jnp.float32)
    h = jnp.maximum(
        jnp.dot(rows, w1_ref[...], preferred_element_type=jnp.float32)
        + b1_ref[...], 0.0)
    out_ref[...] = (
        jnp.dot(h, w2_ref[...], preferred_element_type=jnp.float32)
        + b2_ref[...])


def _build_out_table(ids_p, emb_p, W1, b1, W2, b2):
    return pl.pallas_call(
        _tc_table_mlp,
        out_shape=jax.ShapeDtypeStruct((ROWS, EMB), jnp.float32),
    )(ids_p, emb_p, W1, b1.reshape(1, HID), W2, b2.reshape(1, EMB))


def _sc_gather_kernel(table_hbm, idx_hbm, out_hbm, table_v, idx_v, buf_a,
                      buf_b, sem_a, sem_b):
    wid = lax.axis_index("s") * NC + lax.axis_index("c")
    base = wid * B_PER_W
    obase = base * EMB

    pltpu.sync_copy(table_hbm.at[pl.ds(0, TBL_WORDS)], table_v)
    pltpu.sync_copy(idx_hbm.at[pl.ds(base, B_PER_W)], idx_v.at[pl.ds(0, B_PER_W)])
    idx_v[pl.ds(B_PER_W, 16)] = jnp.zeros((16,), jnp.int32)

    lane = lax.broadcasted_iota(jnp.int32, (16,), 0)
    lane8 = lane & 7
    half = (lane >> 3) * (EMB // 2)
    dst_base = lane8 * EMB + half

    def fill(g, buf):
        r0 = pl.multiple_of(g * GROUP, 8)
        tok = plsc.load_gather(idx_v, [r0 + lane8])
        src_base = tok * EMB + half

        def col_block(cb, carry):
            off = cb * 16
            vals = [plsc.load_gather(table_v, [src_base + (off + ci)])
                    for ci in range(16)]
            for ci in range(16):
                plsc.store_scatter(buf, [dst_base + (off + ci)], vals[ci])
            return carry

        lax.fori_loop(0, EMB // 32, col_block, 0)

    def store(g, buf, sem):
        plsc.subcore_barrier()
        pltpu.async_copy(
            buf, out_hbm.at[pl.ds(obase + g * GROUP * EMB, GROUP * EMB)],
            sem)

    def store_wait(g, buf, sem):
        pltpu.make_async_copy(
            buf, out_hbm.at[pl.ds(obase + g * GROUP * EMB, GROUP * EMB)],
            sem).wait()

    fill(0, buf_a)
    store(0, buf_a, sem_a)
    fill(1, buf_b)
    store(1, buf_b, sem_b)

    def body(p, carry):
        g0 = 2 * p
        store_wait(g0 - 2, buf_a, sem_a)
        plsc.subcore_barrier()
        fill(g0, buf_a)
        store(g0, buf_a, sem_a)
        store_wait(g0 - 1, buf_b, sem_b)
        plsc.subcore_barrier()
        fill(g0 + 1, buf_b)
        store(g0 + 1, buf_b, sem_b)
        return carry

    lax.fori_loop(1, N_GROUPS // 2, body, 0)
    store_wait(N_GROUPS - 2, buf_a, sem_a)
    store_wait(N_GROUPS - 1, buf_b, sem_b)


@functools.cache
def _sc_gather():
    return pl.kernel(
        _sc_gather_kernel,
        out_type=jax.ShapeDtypeStruct((N_TOKENS * EMB,), jnp.float32),
        mesh=plsc.VectorSubcoreMesh(
            core_axis_name="c", subcore_axis_name="s", num_cores=NC,
            num_subcores=NS),
        compiler_params=pltpu.CompilerParams(needs_layout_passes=False),
        scratch_types=[
            pltpu.VMEM((TBL_WORDS,), jnp.float32),
            pltpu.VMEM((B_PER_W + 16,), jnp.int32),
            pltpu.VMEM((GROUP * EMB,), jnp.float32),
            pltpu.VMEM((GROUP * EMB,), jnp.float32),
            pltpu.SemaphoreType.DMA,
            pltpu.SemaphoreType.DMA,
        ],
    )


@jax.jit
def kernel(prompt_token_ids, input_ids, emb_table, W1, b1, W2, b2):
    ids_p = jnp.full((1, ROWS), -1, jnp.int32)
    ids_p = ids_p.at[0, :LENGTH].set(input_ids.astype(jnp.int32))
    emb_p = jnp.pad(emb_table, ((0, ROWS - LENGTH), (0, 0)))
    out_table = _build_out_table(ids_p, emb_p, W1, b1, W2, b2)
    idx = prompt_token_ids.reshape(-1).astype(jnp.int32)
    out_flat = _sc_gather()(out_table.reshape(-1), idx)
    return out_flat.reshape(N_TOKENS, EMB)


# speedup vs baseline: 5.3088x; 5.3088x over previous
import functools

import jax
import jax.numpy as jnp
from jax import lax
from jax.experimental import pallas as pl
from jax.experimental.pallas import tpu as pltpu
from jax.experimental.pallas import tpu_sc as plsc

LENGTH = 100
EMB = 1024
HID = 1024
ROWS = 128

NC = 2
NS = 16
NW = NC * NS

N_TOKENS = 1024 * 50
B_PER_W = N_TOKENS // NW
CHUNK = 40
N_CHUNKS = B_PER_W // CHUNK


def _tc_table_mlp(ids_ref, emb_ref, w1_ref, b1_ref, w2_ref, b2_ref, out_ref):
    ids = ids_ref[...]
    v = lax.broadcasted_iota(jnp.int32, (ROWS, ROWS), 0)
    j = lax.broadcasted_iota(jnp.int32, (ROWS, ROWS), 1)
    match = (ids == v).astype(jnp.float32)
    tri = (lax.broadcasted_iota(jnp.int32, (ROWS, ROWS), 0) <= j).astype(
        jnp.float32)
    c = jnp.dot(match, tri, preferred_element_type=jnp.float32)
    first = match * (c == 1.0).astype(jnp.float32)
    nomatch = (c[:, ROWS - 1:ROWS] == 0.0).astype(jnp.float32)
    col0 = (j == 0).astype(jnp.float32)
    sel = first + nomatch * col0
    rows = jnp.dot(sel, emb_ref[...], preferred_element_type=jnp.float32)
    h = jnp.maximum(
        jnp.dot(rows, w1_ref[...], preferred_element_type=jnp.float32)
        + b1_ref[...], 0.0)
    out_ref[...] = (
        jnp.dot(h, w2_ref[...], preferred_element_type=jnp.float32)
        + b2_ref[...])


def _build_out_table(ids_p, emb_p, W1, b1, W2, b2):
    return pl.pallas_call(
        _tc_table_mlp,
        out_shape=jax.ShapeDtypeStruct((ROWS, EMB), jnp.float32),
    )(ids_p, emb_p, W1, b1.reshape(1, HID), W2, b2.reshape(1, EMB))


def _sc_gather_kernel(table_hbm, idx_hbm, out_hbm, idx_a, idx_b, rows_a,
                      rows_b, gsem_a, gsem_b, ssem_a, ssem_b):
    wid = lax.axis_index("s") * NC + lax.axis_index("c")
    base = wid * B_PER_W

    def fill(g, idx_v, rows_v, gsem):
        pltpu.sync_copy(idx_hbm.at[pl.ds(base + g * CHUNK, CHUNK)], idx_v)
        pltpu.async_copy(table_hbm.at[idx_v], rows_v, gsem)

    def fill_wait(idx_v, rows_v, gsem):
        pltpu.make_async_copy(table_hbm.at[idx_v], rows_v, gsem).wait()

    def store(g, rows_v, ssem):
        pltpu.async_copy(
            rows_v, out_hbm.at[pl.ds(base + g * CHUNK, CHUNK)], ssem)

    def store_wait(g, rows_v, ssem):
        pltpu.make_async_copy(
            rows_v, out_hbm.at[pl.ds(base + g * CHUNK, CHUNK)], ssem).wait()

    fill(0, idx_a, rows_a, gsem_a)
    fill(1, idx_b, rows_b, gsem_b)
    fill_wait(idx_a, rows_a, gsem_a)
    store(0, rows_a, ssem_a)
    fill_wait(idx_b, rows_b, gsem_b)
    store(1, rows_b, ssem_b)

    def body(p, carry):
        g = 2 * p
        store_wait(g - 2, rows_a, ssem_a)
        fill(g, idx_a, rows_a, gsem_a)
        store_wait(g - 1, rows_b, ssem_b)
        fill(g + 1, idx_b, rows_b, gsem_b)
        fill_wait(idx_a, rows_a, gsem_a)
        store(g, rows_a, ssem_a)
        fill_wait(idx_b, rows_b, gsem_b)
        store(g + 1, rows_b, ssem_b)
        return carry

    lax.fori_loop(1, N_CHUNKS // 2, body, 0)
    store_wait(N_CHUNKS - 2, rows_a, ssem_a)
    store_wait(N_CHUNKS - 1, rows_b, ssem_b)


@functools.cache
def _sc_gather():
    return pl.kernel(
        _sc_gather_kernel,
        out_type=jax.ShapeDtypeStruct((N_TOKENS, EMB), jnp.float32),
        mesh=plsc.VectorSubcoreMesh(
            core_axis_name="c", subcore_axis_name="s", num_cores=NC,
            num_subcores=NS),
        compiler_params=pltpu.CompilerParams(needs_layout_passes=False),
        scratch_types=[
            pltpu.VMEM((CHUNK,), jnp.int32),
            pltpu.VMEM((CHUNK,), jnp.int32),
            pltpu.VMEM((CHUNK, EMB), jnp.float32),
            pltpu.VMEM((CHUNK, EMB), jnp.float32),
            pltpu.SemaphoreType.DMA,
            pltpu.SemaphoreType.DMA,
            pltpu.SemaphoreType.DMA,
            pltpu.SemaphoreType.DMA,
        ],
    )


@jax.jit
def kernel(prompt_token_ids, input_ids, emb_table, W1, b1, W2, b2):
    ids_p = jnp.full((1, ROWS), -1, jnp.int32)
    ids_p = ids_p.at[0, :LENGTH].set(input_ids.astype(jnp.int32))
    emb_p = jnp.pad(emb_table, ((0, ROWS - LENGTH), (0, 0)))
    out_table = _build_out_table(ids_p, emb_p, W1, b1, W2, b2)
    idx = prompt_token_ids.reshape(-1).astype(jnp.int32)
    return _sc_gather()(out_table, idx)


# speedup vs baseline: 5.3568x; 1.0090x over previous
import functools

import jax
import jax.numpy as jnp
from jax import lax
from jax.experimental import pallas as pl
from jax.experimental.pallas import tpu as pltpu
from jax.experimental.pallas import tpu_sc as plsc

LENGTH = 100
EMB = 1024
HID = 1024
ROWS = 128

NC = 2
NS = 16
NW = NC * NS

N_TOKENS = 1024 * 50
B_PER_W = N_TOKENS // NW
CHUNK = 40
N_CHUNKS = B_PER_W // CHUNK


def _tc_table_mlp(ids_ref, emb_ref, w1_ref, b1_ref, w2_ref, b2_ref, out_ref):
    ids = ids_ref[...]
    v = lax.broadcasted_iota(jnp.int32, (ROWS, ROWS), 0)
    j = lax.broadcasted_iota(jnp.int32, (ROWS, ROWS), 1)
    match = (ids == v).astype(jnp.float32)
    tri = (lax.broadcasted_iota(jnp.int32, (ROWS, ROWS), 0) <= j).astype(
        jnp.float32)
    c = jnp.dot(match, tri, preferred_element_type=jnp.float32)
    first = match * (c == 1.0).astype(jnp.float32)
    nomatch = (c[:, ROWS - 1:ROWS] == 0.0).astype(jnp.float32)
    col0 = (j == 0).astype(jnp.float32)
    sel = first + nomatch * col0
    rows = jnp.dot(sel, emb_ref[...], preferred_element_type=jnp.float32)
    h = jnp.maximum(
        jnp.dot(rows, w1_ref[...], preferred_element_type=jnp.float32)
        + b1_ref[...], 0.0)
    out_ref[...] = (
        jnp.dot(h, w2_ref[...], preferred_element_type=jnp.float32)
        + b2_ref[...])


def _build_out_table(ids_p, emb_p, W1, b1, W2, b2):
    return pl.pallas_call(
        _tc_table_mlp,
        out_shape=jax.ShapeDtypeStruct((ROWS, EMB), jnp.float32),
    )(ids_p, emb_p, W1, b1.reshape(1, HID), W2, b2.reshape(1, EMB))


def _sc_gather_kernel(table_hbm, idx_hbm, out_hbm, idx_v, rows_a,
                      rows_b, gsem_a, gsem_b, ssem_a, ssem_b):
    wid = lax.axis_index("s") * NC + lax.axis_index("c")
    base = wid * B_PER_W

    pltpu.sync_copy(idx_hbm.at[pl.ds(base, B_PER_W)], idx_v)

    def fill(g, rows_v, gsem):
        pltpu.async_copy(
            table_hbm.at[idx_v.at[pl.ds(g * CHUNK, CHUNK)]], rows_v, gsem)

    def fill_wait(g, rows_v, gsem):
        pltpu.make_async_copy(
            table_hbm.at[idx_v.at[pl.ds(g * CHUNK, CHUNK)]], rows_v,
            gsem).wait()

    def store(g, rows_v, ssem):
        pltpu.async_copy(
            rows_v, out_hbm.at[pl.ds(base + g * CHUNK, CHUNK)], ssem)

    def store_wait(g, rows_v, ssem):
        pltpu.make_async_copy(
            rows_v, out_hbm.at[pl.ds(base + g * CHUNK, CHUNK)], ssem).wait()

    fill(0, rows_a, gsem_a)
    fill(1, rows_b, gsem_b)
    fill_wait(0, rows_a, gsem_a)
    store(0, rows_a, ssem_a)
    fill_wait(1, rows_b, gsem_b)
    store(1, rows_b, ssem_b)

    def body(p, carry):
        g = 2 * p
        store_wait(g - 2, rows_a, ssem_a)
        fill(g, rows_a, gsem_a)
        store_wait(g - 1, rows_b, ssem_b)
        fill(g + 1, rows_b, gsem_b)
        fill_wait(g, rows_a, gsem_a)
        store(g, rows_a, ssem_a)
        fill_wait(g + 1, rows_b, gsem_b)
        store(g + 1, rows_b, ssem_b)
        return carry

    lax.fori_loop(1, N_CHUNKS // 2, body, 0)
    store_wait(N_CHUNKS - 2, rows_a, ssem_a)
    store_wait(N_CHUNKS - 1, rows_b, ssem_b)


@functools.cache
def _sc_gather():
    return pl.kernel(
        _sc_gather_kernel,
        out_type=jax.ShapeDtypeStruct((N_TOKENS, EMB), jnp.float32),
        mesh=plsc.VectorSubcoreMesh(
            core_axis_name="c", subcore_axis_name="s", num_cores=NC,
            num_subcores=NS),
        compiler_params=pltpu.CompilerParams(needs_layout_passes=False),
        scratch_types=[
            pltpu.VMEM((B_PER_W,), jnp.int32),
            pltpu.VMEM((CHUNK, EMB), jnp.float32),
            pltpu.VMEM((CHUNK, EMB), jnp.float32),
            pltpu.SemaphoreType.DMA,
            pltpu.SemaphoreType.DMA,
            pltpu.SemaphoreType.DMA,
            pltpu.SemaphoreType.DMA,
        ],
    )


@jax.jit
def kernel(prompt_token_ids, input_ids, emb_table, W1, b1, W2, b2):
    ids_p = jnp.full((1, ROWS), -1, jnp.int32)
    ids_p = ids_p.at[0, :LENGTH].set(input_ids.astype(jnp.int32))
    emb_p = jnp.pad(emb_table, ((0, ROWS - LENGTH), (0, 0)))
    out_table = _build_out_table(ids_p, emb_p, W1, b1, W2, b2)
    idx = prompt_token_ids.reshape(-1).astype(jnp.int32)
    return _sc_gather()(out_table, idx)


# speedup vs baseline: 8.2803x; 1.5458x over previous
import functools

import jax
import jax.numpy as jnp
from jax import lax
from jax.experimental import pallas as pl
from jax.experimental.pallas import tpu as pltpu
from jax.experimental.pallas import tpu_sc as plsc

LENGTH = 100
EMB = 1024
HID = 1024
ROWS = 128

NC = 2
NS = 16
NW = NC * NS

N_TOKENS = 1024 * 50
B_PER_W = N_TOKENS // NW
CHUNK = 40
N_CHUNKS = B_PER_W // CHUNK


def _tc_table_mlp(ids_ref, emb_ref, w1_ref, b1_ref, w2_ref, b2_ref, out_ref):
    ids = ids_ref[...]
    v = lax.broadcasted_iota(jnp.int32, (ROWS, ROWS), 0)
    j = lax.broadcasted_iota(jnp.int32, (ROWS, ROWS), 1)
    match = (ids == v).astype(jnp.float32)
    tri = (lax.broadcasted_iota(jnp.int32, (ROWS, ROWS), 0) <= j).astype(
        jnp.float32)
    c = jnp.dot(match, tri, preferred_element_type=jnp.float32)
    first = match * (c == 1.0).astype(jnp.float32)
    nomatch = (c[:, ROWS - 1:ROWS] == 0.0).astype(jnp.float32)
    col0 = (j == 0).astype(jnp.float32)
    sel = first + nomatch * col0
    rows = jnp.dot(sel, emb_ref[...], preferred_element_type=jnp.float32)
    h = jnp.maximum(
        jnp.dot(rows, w1_ref[...], preferred_element_type=jnp.float32)
        + b1_ref[...], 0.0)
    out_ref[...] = (
        jnp.dot(h, w2_ref[...], preferred_element_type=jnp.float32)
        + b2_ref[...])


def _build_out_table(ids_p, emb_p, W1, b1, W2, b2):
    return pl.pallas_call(
        _tc_table_mlp,
        out_shape=jax.ShapeDtypeStruct((ROWS, EMB), jnp.float32),
    )(ids_p, emb_p, W1, b1.reshape(1, HID), W2, b2.reshape(1, EMB))


def _sc_gather_kernel(table_hbm, idx_hbm, out_hbm, idx_v, rows_a,
                      rows_b, gsem_a, gsem_b, ssem_a, ssem_b):
    wid = lax.axis_index("s") * NC + lax.axis_index("c")
    base = wid * B_PER_W

    pltpu.sync_copy(idx_hbm.at[pl.ds(base, B_PER_W)], idx_v)

    ofs = wid * ROWS

    def retarget(i, carry):
        s = pl.ds(i * 16, 16)
        idx_v[s] = idx_v[s] + ofs
        return carry

    lax.fori_loop(0, B_PER_W // 16, retarget, 0)

    def fill(g, rows_v, gsem):
        pltpu.async_copy(
            table_hbm.at[idx_v.at[pl.ds(g * CHUNK, CHUNK)]], rows_v, gsem)

    def fill_wait(g, rows_v, gsem):
        pltpu.make_async_copy(
            table_hbm.at[idx_v.at[pl.ds(g * CHUNK, CHUNK)]], rows_v,
            gsem).wait()

    def store(g, rows_v, ssem):
        pltpu.async_copy(
            rows_v, out_hbm.at[pl.ds(base + g * CHUNK, CHUNK)], ssem)

    def store_wait(g, rows_v, ssem):
        pltpu.make_async_copy(
            rows_v, out_hbm.at[pl.ds(base + g * CHUNK, CHUNK)], ssem).wait()

    fill(0, rows_a, gsem_a)
    fill(1, rows_b, gsem_b)
    fill_wait(0, rows_a, gsem_a)
    store(0, rows_a, ssem_a)
    fill_wait(1, rows_b, gsem_b)
    store(1, rows_b, ssem_b)

    def body(p, carry):
        g = 2 * p
        store_wait(g - 2, rows_a, ssem_a)
        fill(g, rows_a, gsem_a)
        store_wait(g - 1, rows_b, ssem_b)
        fill(g + 1, rows_b, gsem_b)
        fill_wait(g, rows_a, gsem_a)
        store(g, rows_a, ssem_a)
        fill_wait(g + 1, rows_b, gsem_b)
        store(g + 1, rows_b, ssem_b)
        return carry

    lax.fori_loop(1, N_CHUNKS // 2, body, 0)
    store_wait(N_CHUNKS - 2, rows_a, ssem_a)
    store_wait(N_CHUNKS - 1, rows_b, ssem_b)


@functools.cache
def _sc_gather():
    return pl.kernel(
        _sc_gather_kernel,
        out_type=jax.ShapeDtypeStruct((N_TOKENS, EMB), jnp.float32),
        mesh=plsc.VectorSubcoreMesh(
            core_axis_name="c", subcore_axis_name="s", num_cores=NC,
            num_subcores=NS),
        compiler_params=pltpu.CompilerParams(needs_layout_passes=False),
        scratch_types=[
            pltpu.VMEM((B_PER_W,), jnp.int32),
            pltpu.VMEM((CHUNK, EMB), jnp.float32),
            pltpu.VMEM((CHUNK, EMB), jnp.float32),
            pltpu.SemaphoreType.DMA,
            pltpu.SemaphoreType.DMA,
            pltpu.SemaphoreType.DMA,
            pltpu.SemaphoreType.DMA,
        ],
    )


@jax.jit
def kernel(prompt_token_ids, input_ids, emb_table, W1, b1, W2, b2):
    ids_p = jnp.full((1, ROWS), -1, jnp.int32)
    ids_p = ids_p.at[0, :LENGTH].set(input_ids.astype(jnp.int32))
    emb_p = jnp.pad(emb_table, ((0, ROWS - LENGTH), (0, 0)))
    out_table = _build_out_table(ids_p, emb_p, W1, b1, W2, b2)
    table_rep = jnp.tile(out_table, (NW, 1))
    idx = prompt_token_ids.reshape(-1).astype(jnp.int32)
    return _sc_gather()(table_rep, idx)


# speedup vs baseline: 8.5145x; 1.0283x over previous
import functools

import jax
import jax.numpy as jnp
from jax import lax
from jax.experimental import pallas as pl
from jax.experimental.pallas import tpu as pltpu
from jax.experimental.pallas import tpu_sc as plsc

LENGTH = 100
EMB = 1024
HID = 1024
ROWS = 128

NC = 2
NS = 16
NW = NC * NS

N_TOKENS = 1024 * 50
B_PER_W = N_TOKENS // NW
CHUNK = 40
N_CHUNKS = B_PER_W // CHUNK


def _tc_table_mlp(ids_ref, emb_ref, w1_ref, b1_ref, w2_ref, b2_ref, out_ref):
    ids = ids_ref[...]
    v = lax.broadcasted_iota(jnp.int32, (ROWS, ROWS), 0)
    j = lax.broadcasted_iota(jnp.int32, (ROWS, ROWS), 1)
    match = (ids == v).astype(jnp.float32)
    tri = (lax.broadcasted_iota(jnp.int32, (ROWS, ROWS), 0) <= j).astype(
        jnp.float32)
    c = jnp.dot(match, tri, preferred_element_type=jnp.float32)
    first = match * (c == 1.0).astype(jnp.float32)
    nomatch = (c[:, ROWS - 1:ROWS] == 0.0).astype(jnp.float32)
    col0 = (j == 0).astype(jnp.float32)
    sel = first + nomatch * col0
    rows = jnp.dot(sel, emb_ref[...], preferred_element_type=jnp.float32)
    h = jnp.maximum(
        jnp.dot(rows, w1_ref[...], preferred_element_type=jnp.float32)
        + b1_ref[...], 0.0)
    out_ref[...] = (
        jnp.dot(h, w2_ref[...], preferred_element_type=jnp.float32)
        + b2_ref[...])


def _build_out_table(ids_p, emb_p, W1, b1, W2, b2):
    return pl.pallas_call(
        _tc_table_mlp,
        out_shape=jax.ShapeDtypeStruct((ROWS, EMB), jnp.float32),
    )(ids_p, emb_p, W1, b1.reshape(1, HID), W2, b2.reshape(1, EMB))


def _sc_gather_kernel(table_hbm, idx_hbm, out_hbm, idx_v, rows_a, rows_b,
                      rows_c, gsem_a, gsem_b, gsem_c, ssem_a, ssem_b, ssem_c):
    wid = lax.axis_index("s") * NC + lax.axis_index("c")
    base = wid * B_PER_W

    pltpu.sync_copy(idx_hbm.at[pl.ds(base, B_PER_W)], idx_v)

    ofs = wid * ROWS

    def retarget(i, carry):
        s = pl.ds(i * 16, 16)
        idx_v[s] = idx_v[s] + ofs
        return carry

    lax.fori_loop(0, B_PER_W // 16, retarget, 0)

    def fill(g, rows_v, gsem):
        pltpu.async_copy(
            table_hbm.at[idx_v.at[pl.ds(g * CHUNK, CHUNK)]], rows_v, gsem)

    def fill_wait(g, rows_v, gsem):
        pltpu.make_async_copy(
            table_hbm.at[idx_v.at[pl.ds(g * CHUNK, CHUNK)]], rows_v,
            gsem).wait()

    def store(g, rows_v, ssem):
        pltpu.async_copy(
            rows_v, out_hbm.at[pl.ds(base + g * CHUNK, CHUNK)], ssem)

    def store_wait(g, rows_v, ssem):
        pltpu.make_async_copy(
            rows_v, out_hbm.at[pl.ds(base + g * CHUNK, CHUNK)], ssem).wait()

    bufs = ((rows_a, gsem_a, ssem_a), (rows_b, gsem_b, ssem_b),
            (rows_c, gsem_c, ssem_c))

    for k in range(3):
        fill(k, *bufs[k][:2])
    for k in range(3):
        fill_wait(k, *bufs[k][:2])
        store(k, bufs[k][0], bufs[k][2])

    def body(p, carry):
        g0 = 3 * p
        for k in range(3):
            rows_v, gsem, ssem = bufs[k]
            store_wait(g0 + k - 3, rows_v, ssem)
            fill(g0 + k, rows_v, gsem)
        for k in range(3):
            rows_v, gsem, ssem = bufs[k]
            fill_wait(g0 + k, rows_v, gsem)
            store(g0 + k, rows_v, ssem)
        return carry

    lax.fori_loop(1, N_CHUNKS // 3, body, 0)
    g_last = N_CHUNKS - 1
    store_wait(g_last - 3, rows_a, ssem_a)
    fill(g_last, rows_a, gsem_a)
    fill_wait(g_last, rows_a, gsem_a)
    store(g_last, rows_a, ssem_a)
    store_wait(g_last - 2, rows_b, ssem_b)
    store_wait(g_last - 1, rows_c, ssem_c)
    store_wait(g_last, rows_a, ssem_a)


@functools.cache
def _sc_gather():
    return pl.kernel(
        _sc_gather_kernel,
        out_type=jax.ShapeDtypeStruct((N_TOKENS, EMB), jnp.float32),
        mesh=plsc.VectorSubcoreMesh(
            core_axis_name="c", subcore_axis_name="s", num_cores=NC,
            num_subcores=NS),
        compiler_params=pltpu.CompilerParams(needs_layout_passes=False),
        scratch_types=[
            pltpu.VMEM((B_PER_W,), jnp.int32),
            pltpu.VMEM((CHUNK, EMB), jnp.float32),
            pltpu.VMEM((CHUNK, EMB), jnp.float32),
            pltpu.VMEM((CHUNK, EMB), jnp.float32),
            pltpu.SemaphoreType.DMA,
            pltpu.SemaphoreType.DMA,
            pltpu.SemaphoreType.DMA,
            pltpu.SemaphoreType.DMA,
            pltpu.SemaphoreType.DMA,
            pltpu.SemaphoreType.DMA,
        ],
    )


@jax.jit
def kernel(prompt_token_ids, input_ids, emb_table, W1, b1, W2, b2):
    ids_p = jnp.full((1, ROWS), -1, jnp.int32)
    ids_p = ids_p.at[0, :LENGTH].set(input_ids.astype(jnp.int32))
    emb_p = jnp.pad(emb_table, ((0, ROWS - LENGTH), (0, 0)))
    out_table = _build_out_table(ids_p, emb_p, W1, b1, W2, b2)
    table_rep = jnp.tile(out_table, (NW, 1))
    idx = prompt_token_ids.reshape(-1).astype(jnp.int32)
    return _sc_gather()(table_rep, idx)
